# Initial kernel scaffold; baseline (speedup 1.0000x reference)
import functools

import jax
import jax.numpy as jnp
from jax import lax
from jax.experimental import pallas as pl
from jax.experimental.pallas import tpu as pltpu
from jax.experimental.pallas import tpu_sc as plsc

N = 10000
E = 320000
D = 128
H = 16
C = 7

NC = 2
NS = 16
NW = NC * NS
NPAD = 10240
RPT = NPAD // NS
EW = 10112
CHUNK = 128
NCHUNK = EW // CHUNK
PADROW = N

_mesh = plsc.VectorSubcoreMesh(
    core_axis_name="c", subcore_axis_name="s", num_cores=NC, num_subcores=NS
)


def _deg_body(dst_hbm, zeros_hbm, out_hbm, dst_v, deg_v):
    c = lax.axis_index("c")
    s = lax.axis_index("s")
    wid = c * NS + s
    pltpu.sync_copy(zeros_hbm, deg_v)
    pltpu.sync_copy(dst_hbm.at[wid], dst_v)
    ones = jnp.full((16,), 1.0, jnp.float32)

    def body(i, carry):
        idx = dst_v[pl.ds(i * 16, 16)]
        plsc.addupdate_scatter(deg_v, [idx], ones)
        return carry

    lax.fori_loop(0, EW // 16, body, 0)
    pltpu.sync_copy(deg_v, out_hbm.at[wid])


@jax.jit
def _deg_kernel(dst_w, zeros_n):
    return pl.kernel(
        _deg_body,
        out_type=jax.ShapeDtypeStruct((NW, NPAD), jnp.float32),
        mesh=_mesh,
        scratch_types=[
            pltpu.VMEM((EW,), jnp.int32),
            pltpu.VMEM((NPAD,), jnp.float32),
        ],
    )(dst_w.reshape(NW, EW), zeros_n)


def _msg_body(y_hbm, src_hbm, dst_hbm, zeros_hbm, out_hbm,
              src_v, dst_v, rows_v, y_sp, acc_sp, sem):
    c = lax.axis_index("c")
    s = lax.axis_index("s")
    wid = c * NS + s
    r0 = s * RPT
    pltpu.sync_copy(y_hbm.at[pl.ds(r0, RPT)], y_sp.at[pl.ds(r0, RPT)])
    pltpu.sync_copy(zeros_hbm.at[pl.ds(r0, RPT)], acc_sp.at[pl.ds(r0, RPT)])
    pltpu.sync_copy(src_hbm.at[wid], src_v)
    pltpu.sync_copy(dst_hbm.at[wid], dst_v)
    plsc.subcore_barrier()

    def body(j, carry):
        pltpu.async_copy(y_sp.at[src_v.at[j]], rows_v, sem).wait()
        pltpu.sync_copy(rows_v, acc_sp.at[dst_v.at[j]], add=True)
        return carry

    lax.fori_loop(0, NCHUNK, body, 0)
    plsc.subcore_barrier()
    pltpu.sync_copy(acc_sp.at[pl.ds(r0, RPT)], out_hbm.at[c, pl.ds(r0, RPT)])


@jax.jit
def _msg_kernel(y, src_w, dst_w, zeros_nf):
    return pl.kernel(
        _msg_body,
        out_type=jax.ShapeDtypeStruct((NC, NPAD, H), jnp.float32),
        mesh=_mesh,
        scratch_types=[
            pltpu.VMEM((NCHUNK, CHUNK), jnp.int32),
            pltpu.VMEM((NCHUNK, CHUNK), jnp.int32),
            pltpu.VMEM((CHUNK, H), jnp.float32),
            pltpu.VMEM_SHARED((NPAD, H), jnp.float32),
            pltpu.VMEM_SHARED((NPAD, H), jnp.float32),
            pltpu.SemaphoreType.DMA,
        ],
    )(y, src_w, dst_w, zeros_nf)


BN = 1024


def _prep_body(deg_ref, x_ref, w1_ref, dinv_ref, y1_ref):
    deg = jnp.sum(deg_ref[...], axis=0) + 1.0
    dinv = lax.rsqrt(deg)[:, None]
    dinv_ref[...] = dinv
    h = jnp.dot(x_ref[...], w1_ref[...], preferred_element_type=jnp.float32)
    y1_ref[...] = dinv * h


@jax.jit
def _prep_kernel(deg_parts, xpad, W1):
    return pl.pallas_call(
        _prep_body,
        grid=(NPAD // BN,),
        in_specs=[
            pl.BlockSpec((NW, BN), lambda i: (0, i)),
            pl.BlockSpec((BN, D), lambda i: (i, 0)),
            pl.BlockSpec((D, H), lambda i: (0, 0)),
        ],
        out_specs=[
            pl.BlockSpec((BN, 1), lambda i: (i, 0)),
            pl.BlockSpec((BN, H), lambda i: (i, 0)),
        ],
        out_shape=[
            jax.ShapeDtypeStruct((NPAD, 1), jnp.float32),
            jax.ShapeDtypeStruct((NPAD, H), jnp.float32),
        ],
    )(deg_parts, xpad, W1)


def _mid_body(acc_ref, y1_ref, dinv_ref, w2_ref, b1_ref, y2_ref):
    i = pl.program_id(0)
    a = acc_ref[0] + acc_ref[1] + y1_ref[...]
    dinv = dinv_ref[...]
    out1 = jnp.maximum(dinv * a + b1_ref[...], 0.0)
    h2 = jnp.dot(out1, w2_ref[...], preferred_element_type=jnp.float32)
    y2 = dinv * h2
    rows = i * BN + lax.broadcasted_iota(jnp.int32, (BN, H), 0)
    y2_ref[...] = jnp.where(rows < N, y2, 0.0)


@jax.jit
def _mid_kernel(acc1, y1, dinv, W2p, b1r):
    return pl.pallas_call(
        _mid_body,
        grid=(NPAD // BN,),
        in_specs=[
            pl.BlockSpec((NC, BN, H), lambda i: (0, i, 0)),
            pl.BlockSpec((BN, H), lambda i: (i, 0)),
            pl.BlockSpec((BN, 1), lambda i: (i, 0)),
            pl.BlockSpec((H, H), lambda i: (0, 0)),
            pl.BlockSpec((1, H), lambda i: (0, 0)),
        ],
        out_specs=pl.BlockSpec((BN, H), lambda i: (i, 0)),
        out_shape=jax.ShapeDtypeStruct((NPAD, H), jnp.float32),
    )(acc1, y1, dinv, W2p, b1r)


BF = 1000


def _fin_body(acc_ref, y2_ref, dinv_ref, b2_ref, out_ref):
    a = acc_ref[0] + acc_ref[1] + y2_ref[...]
    t = dinv_ref[...] * a + b2_ref[...]
    logits = t[:, :C]
    m = jnp.max(logits, axis=1, keepdims=True)
    ex = jnp.exp(logits - m)
    lse = jnp.log(jnp.sum(ex, axis=1, keepdims=True)) + m
    out_ref[...] = logits - lse


@jax.jit
def _fin_kernel(acc2, y2, dinv, b2p):
    return pl.pallas_call(
        _fin_body,
        grid=(N // BF,),
        in_specs=[
            pl.BlockSpec((NC, BF, H), lambda i: (0, i, 0)),
            pl.BlockSpec((BF, H), lambda i: (i, 0)),
            pl.BlockSpec((BF, 1), lambda i: (i, 0)),
            pl.BlockSpec((1, H), lambda i: (0, 0)),
        ],
        out_specs=pl.BlockSpec((BF, C), lambda i: (i, 0)),
        out_shape=jax.ShapeDtypeStruct((N, C), jnp.float32),
    )(acc2, y2, dinv, b2p)


def kernel(x, edge_index, W1, b1, W2, b2):
    pad = NW * EW - E
    src_w = jnp.concatenate(
        [edge_index[0], jnp.full((pad,), PADROW, jnp.int32)]).reshape(NW, NCHUNK, CHUNK)
    dst_w = jnp.concatenate(
        [edge_index[1], jnp.full((pad,), PADROW, jnp.int32)]).reshape(NW, NCHUNK, CHUNK)
    xpad = jnp.pad(x, ((0, NPAD - N), (0, 0)))
    W2p = jnp.pad(W2, ((0, 0), (0, H - C)))
    b1r = b1.reshape(1, H)
    b2p = jnp.pad(b2, (0, H - C)).reshape(1, H)
    zeros_n = jnp.zeros((NPAD,), jnp.float32)
    zeros_nf = jnp.zeros((NPAD, H), jnp.float32)

    deg_parts = _deg_kernel(dst_w, zeros_n)
    dinv, y1 = _prep_kernel(deg_parts, xpad, W1)
    acc1 = _msg_kernel(y1, src_w, dst_w, zeros_nf)
    y2 = _mid_kernel(acc1, y1, dinv, W2p, b1r)
    acc2 = _msg_kernel(y2, src_w, dst_w, zeros_nf)
    return _fin_kernel(acc2, y2, dinv, b2p)


# speedup vs baseline: 30.3324x; 30.3324x over previous
import functools

import jax
import jax.numpy as jnp
from jax import lax
from jax.experimental import pallas as pl
from jax.experimental.pallas import tpu as pltpu
from jax.experimental.pallas import tpu_sc as plsc

N = 10000
E = 320000
D = 128
H = 16
C = 7

NC = 2
NS = 16
NW = NC * NS
NPAD = 10240
RPT = NPAD // NS
EW = 10112
CHUNK = 128
NCHUNK = EW // CHUNK
PADROW = N

_mesh = plsc.VectorSubcoreMesh(
    core_axis_name="c", subcore_axis_name="s", num_cores=NC, num_subcores=NS
)


def _deg_body(dst_hbm, zeros_hbm, out_hbm, dst_v, deg_v):
    c = lax.axis_index("c")
    s = lax.axis_index("s")
    wid = c * NS + s
    pltpu.sync_copy(zeros_hbm, deg_v)
    pltpu.sync_copy(dst_hbm.at[wid], dst_v)
    ones = jnp.full((16,), 1.0, jnp.float32)

    def body(i, carry):
        idx = dst_v[pl.ds(i * 16, 16)]
        plsc.addupdate_scatter(deg_v, [idx], ones)
        return carry

    lax.fori_loop(0, EW // 16, body, 0)
    pltpu.sync_copy(deg_v, out_hbm.at[wid])


@jax.jit
def _deg_kernel(dst_w, zeros_n):
    return pl.kernel(
        _deg_body,
        out_type=jax.ShapeDtypeStruct((NW, NPAD), jnp.float32),
        mesh=_mesh,
        scratch_types=[
            pltpu.VMEM((EW,), jnp.int32),
            pltpu.VMEM((NPAD,), jnp.float32),
        ],
        compiler_params=pltpu.CompilerParams(
            needs_layout_passes=False, use_tc_tiling_on_sc=False),
    )(dst_w.reshape(NW, EW), zeros_n)


def _msg_body(y_hbm, src_hbm, dst_hbm, zeros_hbm, out_hbm,
              sidx_v, didx_v, rows_v, y_sp, acc_sp, sem):
    c = lax.axis_index("c")
    s = lax.axis_index("s")
    wid = c * NS + s
    r0 = s * RPT
    pltpu.sync_copy(y_hbm.at[pl.ds(r0, RPT)], y_sp.at[pl.ds(r0, RPT)])
    pltpu.sync_copy(zeros_hbm.at[pl.ds(r0, RPT)], acc_sp.at[pl.ds(r0, RPT)])
    plsc.subcore_barrier()

    def body(j, carry):
        pltpu.sync_copy(src_hbm.at[wid, j], sidx_v)
        pltpu.sync_copy(dst_hbm.at[wid, j], didx_v)
        pltpu.async_copy(y_sp.at[sidx_v], rows_v, sem).wait()
        pltpu.sync_copy(rows_v, acc_sp.at[didx_v], add=True)
        return carry

    lax.fori_loop(0, NCHUNK, body, 0)
    plsc.subcore_barrier()
    pltpu.sync_copy(acc_sp.at[pl.ds(r0, RPT)], out_hbm.at[c, pl.ds(r0, RPT)])


@jax.jit
def _msg_kernel(y, src_w, dst_w, zeros_nf):
    return pl.kernel(
        _msg_body,
        out_type=jax.ShapeDtypeStruct((NC, NPAD, H), jnp.float32),
        mesh=_mesh,
        scratch_types=[
            pltpu.VMEM((CHUNK,), jnp.int32),
            pltpu.VMEM((CHUNK,), jnp.int32),
            pltpu.VMEM((CHUNK, H), jnp.float32),
            pltpu.VMEM_SHARED((NPAD, H), jnp.float32),
            pltpu.VMEM_SHARED((NPAD, H), jnp.float32),
            pltpu.SemaphoreType.DMA,
        ],
        compiler_params=pltpu.CompilerParams(
            needs_layout_passes=False, use_tc_tiling_on_sc=False),
    )(y, src_w, dst_w, zeros_nf)


BN = 1024


def _prep_body(deg_ref, x_ref, w1_ref, dinv_ref, y1_ref):
    deg = jnp.sum(deg_ref[...], axis=0) + 1.0
    dinv = lax.rsqrt(deg)[:, None]
    dinv_ref[...] = dinv
    h = jnp.dot(x_ref[...], w1_ref[...], preferred_element_type=jnp.float32)
    y1_ref[...] = dinv * h


@jax.jit
def _prep_kernel(deg_parts, xpad, W1):
    return pl.pallas_call(
        _prep_body,
        grid=(NPAD // BN,),
        in_specs=[
            pl.BlockSpec((NW, BN), lambda i: (0, i)),
            pl.BlockSpec((BN, D), lambda i: (i, 0)),
            pl.BlockSpec((D, H), lambda i: (0, 0)),
        ],
        out_specs=[
            pl.BlockSpec((BN, 1), lambda i: (i, 0)),
            pl.BlockSpec((BN, H), lambda i: (i, 0)),
        ],
        out_shape=[
            jax.ShapeDtypeStruct((NPAD, 1), jnp.float32),
            jax.ShapeDtypeStruct((NPAD, H), jnp.float32),
        ],
    )(deg_parts, xpad, W1)


def _mid_body(acc_ref, y1_ref, dinv_ref, w2_ref, b1_ref, y2_ref):
    i = pl.program_id(0)
    a = acc_ref[0] + acc_ref[1] + y1_ref[...]
    dinv = dinv_ref[...]
    out1 = jnp.maximum(dinv * a + b1_ref[...], 0.0)
    h2 = jnp.dot(out1, w2_ref[...], preferred_element_type=jnp.float32)
    y2 = dinv * h2
    rows = i * BN + lax.broadcasted_iota(jnp.int32, (BN, H), 0)
    y2_ref[...] = jnp.where(rows < N, y2, 0.0)


@jax.jit
def _mid_kernel(acc1, y1, dinv, W2p, b1r):
    return pl.pallas_call(
        _mid_body,
        grid=(NPAD // BN,),
        in_specs=[
            pl.BlockSpec((NC, BN, H), lambda i: (0, i, 0)),
            pl.BlockSpec((BN, H), lambda i: (i, 0)),
            pl.BlockSpec((BN, 1), lambda i: (i, 0)),
            pl.BlockSpec((H, H), lambda i: (0, 0)),
            pl.BlockSpec((1, H), lambda i: (0, 0)),
        ],
        out_specs=pl.BlockSpec((BN, H), lambda i: (i, 0)),
        out_shape=jax.ShapeDtypeStruct((NPAD, H), jnp.float32),
    )(acc1, y1, dinv, W2p, b1r)


BF = 1000


def _fin_body(acc_ref, y2_ref, dinv_ref, b2_ref, out_ref):
    a = acc_ref[0] + acc_ref[1] + y2_ref[...]
    t = dinv_ref[...] * a + b2_ref[...]
    logits = t[:, :C]
    m = jnp.max(logits, axis=1, keepdims=True)
    ex = jnp.exp(logits - m)
    lse = jnp.log(jnp.sum(ex, axis=1, keepdims=True)) + m
    out_ref[...] = logits - lse


@jax.jit
def _fin_kernel(acc2, y2, dinv, b2p):
    return pl.pallas_call(
        _fin_body,
        grid=(N // BF,),
        in_specs=[
            pl.BlockSpec((NC, BF, H), lambda i: (0, i, 0)),
            pl.BlockSpec((BF, H), lambda i: (i, 0)),
            pl.BlockSpec((BF, 1), lambda i: (i, 0)),
            pl.BlockSpec((1, H), lambda i: (0, 0)),
        ],
        out_specs=pl.BlockSpec((BF, C), lambda i: (i, 0)),
        out_shape=jax.ShapeDtypeStruct((N, C), jnp.float32),
    )(acc2, y2, dinv, b2p)


def kernel(x, edge_index, W1, b1, W2, b2):
    pad = NW * EW - E
    src_w = jnp.concatenate(
        [edge_index[0], jnp.full((pad,), PADROW, jnp.int32)]).reshape(NW, NCHUNK, CHUNK)
    dst_w = jnp.concatenate(
        [edge_index[1], jnp.full((pad,), PADROW, jnp.int32)]).reshape(NW, NCHUNK, CHUNK)
    xpad = jnp.pad(x, ((0, NPAD - N), (0, 0)))
    W2p = jnp.pad(W2, ((0, 0), (0, H - C)))
    b1r = b1.reshape(1, H)
    b2p = jnp.pad(b2, (0, H - C)).reshape(1, H)
    zeros_n = jnp.zeros((NPAD,), jnp.float32)
    zeros_nf = jnp.zeros((NPAD, H), jnp.float32)

    deg_parts = _deg_kernel(dst_w, zeros_n)
    dinv, y1 = _prep_kernel(deg_parts, xpad, W1)
    acc1 = _msg_kernel(y1, src_w, dst_w, zeros_nf)
    y2 = _mid_kernel(acc1, y1, dinv, W2p, b1r)
    acc2 = _msg_kernel(y2, src_w, dst_w, zeros_nf)
    return _fin_kernel(acc2, y2, dinv, b2p)


# speedup vs baseline: 60.9306x; 2.0088x over previous
import functools

import jax
import jax.numpy as jnp
from jax import lax
from jax.experimental import pallas as pl
from jax.experimental.pallas import tpu as pltpu
from jax.experimental.pallas import tpu_sc as plsc

N = 10000
E = 320000
D = 128
H = 16
C = 7

NC = 2
NS = 16
NW = NC * NS
NPAD = 10240
RPT = NPAD // NS
EW = 10112
CHUNK = 128
NCHUNK = EW // CHUNK
PADROW = N

_mesh = plsc.VectorSubcoreMesh(
    core_axis_name="c", subcore_axis_name="s", num_cores=NC, num_subcores=NS
)


def _deg_body(dst_hbm, zeros_hbm, out_hbm, dst_v, deg_v):
    c = lax.axis_index("c")
    s = lax.axis_index("s")
    wid = c * NS + s
    pltpu.sync_copy(zeros_hbm, deg_v)
    pltpu.sync_copy(dst_hbm.at[wid], dst_v)
    ones = jnp.full((16,), 1.0, jnp.float32)

    def body(i, carry):
        idx = dst_v[pl.ds(i * 16, 16)]
        plsc.addupdate_scatter(deg_v, [idx], ones)
        return carry

    lax.fori_loop(0, EW // 16, body, 0)
    pltpu.sync_copy(deg_v, out_hbm.at[wid])


@jax.jit
def _deg_kernel(dst_w, zeros_n):
    return pl.kernel(
        _deg_body,
        out_type=jax.ShapeDtypeStruct((NW, NPAD), jnp.float32),
        mesh=_mesh,
        scratch_types=[
            pltpu.VMEM((EW,), jnp.int32),
            pltpu.VMEM((NPAD,), jnp.float32),
        ],
        compiler_params=pltpu.CompilerParams(
            needs_layout_passes=False, use_tc_tiling_on_sc=False),
    )(dst_w.reshape(NW, EW), zeros_n)


NBUF = 4
PREF = 2


def _msg_body(y_hbm, src_hbm, dst_hbm, zeros_hbm, out_hbm,
              src_v, dst_v, rows_v, y_sp, acc_sp, gsem, ssem):
    c = lax.axis_index("c")
    s = lax.axis_index("s")
    wid = c * NS + s
    r0 = s * RPT
    pltpu.sync_copy(y_hbm.at[pl.ds(r0, RPT)], y_sp.at[pl.ds(r0, RPT)])
    pltpu.sync_copy(zeros_hbm.at[pl.ds(r0, RPT)], acc_sp.at[pl.ds(r0, RPT)])
    pltpu.sync_copy(src_hbm.at[wid], src_v)
    pltpu.sync_copy(dst_hbm.at[wid], dst_v)
    plsc.subcore_barrier()

    def gather(j, b):
        pltpu.async_copy(y_sp.at[src_v.at[j]], rows_v.at[b], gsem.at[b])

    def scatter(j, b):
        pltpu.async_copy(rows_v.at[b], acc_sp.at[dst_v.at[j]], ssem.at[b],
                         add=True)

    for jp in range(PREF):
        gather(jp, jp % NBUF)

    def body(j, carry):
        b = lax.rem(j, NBUF)
        pltpu.make_async_copy(y_sp.at[src_v.at[j]], rows_v.at[b],
                              gsem.at[b]).wait()
        scatter(j, b)
        jn = j + PREF
        bn = lax.rem(jn, NBUF)

        @pl.when(jn < NCHUNK)
        def _():
            @pl.when(jn >= NBUF)
            def _():
                pltpu.make_async_copy(
                    rows_v.at[bn], acc_sp.at[dst_v.at[jn - NBUF]],
                    ssem.at[bn]).wait()
            gather(jn, bn)

        return carry

    lax.fori_loop(0, NCHUNK, body, 0)
    for j in range(NCHUNK - NBUF, NCHUNK):
        b = j % NBUF
        pltpu.make_async_copy(rows_v.at[b], acc_sp.at[dst_v.at[j]],
                              ssem.at[b]).wait()
    plsc.subcore_barrier()
    pltpu.sync_copy(acc_sp.at[pl.ds(r0, RPT)], out_hbm.at[c, pl.ds(r0, RPT)])


@jax.jit
def _msg_kernel(y, src_w, dst_w, zeros_nf):
    return pl.kernel(
        _msg_body,
        out_type=jax.ShapeDtypeStruct((NC, NPAD, H), jnp.float32),
        mesh=_mesh,
        scratch_types=[
            pltpu.VMEM((NCHUNK, CHUNK), jnp.int32),
            pltpu.VMEM((NCHUNK, CHUNK), jnp.int32),
            pltpu.VMEM((NBUF, CHUNK, H), jnp.float32),
            pltpu.VMEM_SHARED((NPAD, H), jnp.float32),
            pltpu.VMEM_SHARED((NPAD, H), jnp.float32),
            pltpu.SemaphoreType.DMA((NBUF,)),
            pltpu.SemaphoreType.DMA((NBUF,)),
        ],
        compiler_params=pltpu.CompilerParams(
            needs_layout_passes=False, use_tc_tiling_on_sc=False),
    )(y, src_w, dst_w, zeros_nf)


BN = 1024


def _prep_body(deg_ref, x_ref, w1_ref, dinv_ref, y1_ref):
    deg = jnp.sum(deg_ref[...], axis=0) + 1.0
    dinv = lax.rsqrt(deg)[:, None]
    dinv_ref[...] = dinv
    h = jnp.dot(x_ref[...], w1_ref[...], preferred_element_type=jnp.float32)
    y1_ref[...] = dinv * h


@jax.jit
def _prep_kernel(deg_parts, xpad, W1):
    return pl.pallas_call(
        _prep_body,
        grid=(NPAD // BN,),
        in_specs=[
            pl.BlockSpec((NW, BN), lambda i: (0, i)),
            pl.BlockSpec((BN, D), lambda i: (i, 0)),
            pl.BlockSpec((D, H), lambda i: (0, 0)),
        ],
        out_specs=[
            pl.BlockSpec((BN, 1), lambda i: (i, 0)),
            pl.BlockSpec((BN, H), lambda i: (i, 0)),
        ],
        out_shape=[
            jax.ShapeDtypeStruct((NPAD, 1), jnp.float32),
            jax.ShapeDtypeStruct((NPAD, H), jnp.float32),
        ],
    )(deg_parts, xpad, W1)


def _mid_body(acc_ref, y1_ref, dinv_ref, w2_ref, b1_ref, y2_ref):
    i = pl.program_id(0)
    a = acc_ref[0] + acc_ref[1] + y1_ref[...]
    dinv = dinv_ref[...]
    out1 = jnp.maximum(dinv * a + b1_ref[...], 0.0)
    h2 = jnp.dot(out1, w2_ref[...], preferred_element_type=jnp.float32)
    y2 = dinv * h2
    rows = i * BN + lax.broadcasted_iota(jnp.int32, (BN, H), 0)
    y2_ref[...] = jnp.where(rows < N, y2, 0.0)


@jax.jit
def _mid_kernel(acc1, y1, dinv, W2p, b1r):
    return pl.pallas_call(
        _mid_body,
        grid=(NPAD // BN,),
        in_specs=[
            pl.BlockSpec((NC, BN, H), lambda i: (0, i, 0)),
            pl.BlockSpec((BN, H), lambda i: (i, 0)),
            pl.BlockSpec((BN, 1), lambda i: (i, 0)),
            pl.BlockSpec((H, H), lambda i: (0, 0)),
            pl.BlockSpec((1, H), lambda i: (0, 0)),
        ],
        out_specs=pl.BlockSpec((BN, H), lambda i: (i, 0)),
        out_shape=jax.ShapeDtypeStruct((NPAD, H), jnp.float32),
    )(acc1, y1, dinv, W2p, b1r)


BF = 1000


def _fin_body(acc_ref, y2_ref, dinv_ref, b2_ref, out_ref):
    a = acc_ref[0] + acc_ref[1] + y2_ref[...]
    t = dinv_ref[...] * a + b2_ref[...]
    logits = t[:, :C]
    m = jnp.max(logits, axis=1, keepdims=True)
    ex = jnp.exp(logits - m)
    lse = jnp.log(jnp.sum(ex, axis=1, keepdims=True)) + m
    out_ref[...] = logits - lse


@jax.jit
def _fin_kernel(acc2, y2, dinv, b2p):
    return pl.pallas_call(
        _fin_body,
        grid=(N // BF,),
        in_specs=[
            pl.BlockSpec((NC, BF, H), lambda i: (0, i, 0)),
            pl.BlockSpec((BF, H), lambda i: (i, 0)),
            pl.BlockSpec((BF, 1), lambda i: (i, 0)),
            pl.BlockSpec((1, H), lambda i: (0, 0)),
        ],
        out_specs=pl.BlockSpec((BF, C), lambda i: (i, 0)),
        out_shape=jax.ShapeDtypeStruct((N, C), jnp.float32),
    )(acc2, y2, dinv, b2p)


def kernel(x, edge_index, W1, b1, W2, b2):
    pad = NW * EW - E
    src_w = jnp.concatenate(
        [edge_index[0], jnp.full((pad,), PADROW, jnp.int32)]).reshape(NW, NCHUNK, CHUNK)
    dst_w = jnp.concatenate(
        [edge_index[1], jnp.full((pad,), PADROW, jnp.int32)]).reshape(NW, NCHUNK, CHUNK)
    xpad = jnp.pad(x, ((0, NPAD - N), (0, 0)))
    W2p = jnp.pad(W2, ((0, 0), (0, H - C)))
    b1r = b1.reshape(1, H)
    b2p = jnp.pad(b2, (0, H - C)).reshape(1, H)
    zeros_n = jnp.zeros((NPAD,), jnp.float32)
    zeros_nf = jnp.zeros((NPAD, H), jnp.float32)

    deg_parts = _deg_kernel(dst_w, zeros_n)
    dinv, y1 = _prep_kernel(deg_parts, xpad, W1)
    acc1 = _msg_kernel(y1, src_w, dst_w, zeros_nf)
    y2 = _mid_kernel(acc1, y1, dinv, W2p, b1r)
    acc2 = _msg_kernel(y2, src_w, dst_w, zeros_nf)
    return _fin_kernel(acc2, y2, dinv, b2p)


# speedup vs baseline: 64.6176x; 1.0605x over previous
import jax
import jax.numpy as jnp
from jax import lax
from jax.experimental import pallas as pl
from jax.experimental.pallas import tpu as pltpu
from jax.experimental.pallas import tpu_sc as plsc

N = 10000
E = 320000
D = 128
H = 16
C = 7

NC = 2
NS = 16
NW = NC * NS
RPT = N // NS
EW = E // NW
CHUNK = 128
NCHUNK = EW // CHUNK
TAIL = EW - NCHUNK * CHUNK
NBUF = 4
PREF = 2
ZR = 128

_mesh = plsc.VectorSubcoreMesh(
    core_axis_name="c", subcore_axis_name="s", num_cores=NC, num_subcores=NS
)
_sc_params = pltpu.CompilerParams(
    needs_layout_passes=False, use_tc_tiling_on_sc=False)


def _deg_body(dst_hbm, out_hbm, dst_v, deg_v):
    c = lax.axis_index("c")
    s = lax.axis_index("s")
    wid = c * NS + s
    pltpu.sync_copy(dst_hbm.at[wid], dst_v)
    zero = jnp.zeros((16,), jnp.float32)

    def zbody(i, carry):
        deg_v[pl.ds(i * 16, 16)] = zero
        return carry

    lax.fori_loop(0, N // 16, zbody, 0)
    ones = jnp.full((16,), 1.0, jnp.float32)

    def body(i, carry):
        for u in range(5):
            idx = dst_v[pl.ds(i * 80 + u * 16, 16)]
            plsc.addupdate_scatter(deg_v, [idx], ones)
        return carry

    lax.fori_loop(0, EW // 80, body, 0)
    pltpu.sync_copy(deg_v, out_hbm.at[wid])


@jax.jit
def _deg_kernel(dst_w):
    return pl.kernel(
        _deg_body,
        out_type=jax.ShapeDtypeStruct((NW, N), jnp.float32),
        mesh=_mesh,
        scratch_types=[
            pltpu.VMEM((EW,), jnp.int32),
            pltpu.VMEM((N,), jnp.float32),
        ],
        compiler_params=_sc_params,
    )(dst_w)


def _msg_body(y_hbm, src_hbm, dst_hbm, out_hbm,
              src_v, dst_v, rows_v, zbuf_v, y_sp, acc_sp, gsem, ssem):
    c = lax.axis_index("c")
    s = lax.axis_index("s")
    wid = c * NS + s
    r0 = s * RPT
    pltpu.sync_copy(y_hbm.at[pl.ds(r0, RPT)], y_sp.at[pl.ds(r0, RPT)])
    zero = jnp.zeros((16,), jnp.float32)

    def zbody(i, carry):
        zbuf_v[i] = zero
        return carry

    lax.fori_loop(0, ZR, zbody, 0)
    for q in range(4):
        pltpu.sync_copy(zbuf_v, acc_sp.at[pl.ds(r0 + q * ZR, ZR)])
    pltpu.sync_copy(zbuf_v.at[pl.ds(0, RPT - 4 * ZR)],
                    acc_sp.at[pl.ds(r0 + 4 * ZR, RPT - 4 * ZR)])
    pltpu.sync_copy(src_hbm.at[wid], src_v)
    pltpu.sync_copy(dst_hbm.at[wid], dst_v)
    plsc.subcore_barrier()

    def gather(j, b):
        pltpu.async_copy(y_sp.at[src_v.at[pl.ds(j * CHUNK, CHUNK)]],
                         rows_v.at[b], gsem.at[b])

    def scatter(j, b):
        pltpu.async_copy(rows_v.at[b],
                         acc_sp.at[dst_v.at[pl.ds(j * CHUNK, CHUNK)]],
                         ssem.at[b], add=True)

    def wait_gather(j, b):
        pltpu.make_async_copy(y_sp.at[src_v.at[pl.ds(j * CHUNK, CHUNK)]],
                              rows_v.at[b], gsem.at[b]).wait()

    def wait_scatter(j, b):
        pltpu.make_async_copy(rows_v.at[b],
                              acc_sp.at[dst_v.at[pl.ds(j * CHUNK, CHUNK)]],
                              ssem.at[b]).wait()

    for jp in range(PREF):
        gather(jp, jp % NBUF)

    def body(j, carry):
        b = lax.rem(j, NBUF)
        wait_gather(j, b)
        scatter(j, b)
        jn = j + PREF
        bn = lax.rem(jn, NBUF)

        @pl.when(jn < NCHUNK)
        def _():
            @pl.when(jn >= NBUF)
            def _():
                wait_scatter(jn - NBUF, bn)
            gather(jn, bn)

        return carry

    lax.fori_loop(0, NCHUNK, body, 0)
    for j in range(NCHUNK - NBUF, NCHUNK):
        wait_scatter(j, j % NBUF)
    t0 = NCHUNK * CHUNK
    pltpu.async_copy(y_sp.at[src_v.at[pl.ds(t0, TAIL)]],
                     rows_v.at[0, pl.ds(0, TAIL)], gsem.at[0])
    pltpu.make_async_copy(y_sp.at[src_v.at[pl.ds(t0, TAIL)]],
                          rows_v.at[0, pl.ds(0, TAIL)], gsem.at[0]).wait()
    pltpu.sync_copy(rows_v.at[0, pl.ds(0, TAIL)],
                    acc_sp.at[dst_v.at[pl.ds(t0, TAIL)]], add=True)
    plsc.subcore_barrier()
    pltpu.sync_copy(acc_sp.at[pl.ds(r0, RPT)], out_hbm.at[c, pl.ds(r0, RPT)])


@jax.jit
def _msg_kernel(y, src_w, dst_w):
    return pl.kernel(
        _msg_body,
        out_type=jax.ShapeDtypeStruct((NC, N, H), jnp.float32),
        mesh=_mesh,
        scratch_types=[
            pltpu.VMEM((EW,), jnp.int32),
            pltpu.VMEM((EW,), jnp.int32),
            pltpu.VMEM((NBUF, CHUNK, H), jnp.float32),
            pltpu.VMEM((ZR, H), jnp.float32),
            pltpu.VMEM_SHARED((N, H), jnp.float32),
            pltpu.VMEM_SHARED((N, H), jnp.float32),
            pltpu.SemaphoreType.DMA((NBUF,)),
            pltpu.SemaphoreType.DMA((NBUF,)),
        ],
        compiler_params=_sc_params,
    )(y, src_w, dst_w)


BN = 1024


def _prep_body(deg_ref, x_ref, w1_ref, dinv_ref, y1_ref):
    deg = jnp.sum(deg_ref[...], axis=0) + 1.0
    dinv = lax.rsqrt(deg)[:, None]
    dinv_ref[...] = dinv
    h = jnp.dot(x_ref[...], w1_ref[...], preferred_element_type=jnp.float32)
    y1_ref[...] = dinv * h


@jax.jit
def _prep_kernel(deg_parts, x, W1):
    return pl.pallas_call(
        _prep_body,
        grid=(pl.cdiv(N, BN),),
        in_specs=[
            pl.BlockSpec((NW, BN), lambda i: (0, i)),
            pl.BlockSpec((BN, D), lambda i: (i, 0)),
            pl.BlockSpec((D, H), lambda i: (0, 0)),
        ],
        out_specs=[
            pl.BlockSpec((BN, 1), lambda i: (i, 0)),
            pl.BlockSpec((BN, H), lambda i: (i, 0)),
        ],
        out_shape=[
            jax.ShapeDtypeStruct((N, 1), jnp.float32),
            jax.ShapeDtypeStruct((N, H), jnp.float32),
        ],
    )(deg_parts, x, W1)


def _mid_body(acc_ref, y1_ref, dinv_ref, w2_ref, b1_ref, y2_ref):
    a = acc_ref[0] + acc_ref[1] + y1_ref[...]
    dinv = dinv_ref[...]
    out1 = jnp.maximum(dinv * a + b1_ref[...], 0.0)
    h2 = jnp.dot(out1, w2_ref[...], preferred_element_type=jnp.float32)
    y2_ref[...] = dinv * h2


@jax.jit
def _mid_kernel(acc1, y1, dinv, W2p, b1r):
    return pl.pallas_call(
        _mid_body,
        grid=(pl.cdiv(N, BN),),
        in_specs=[
            pl.BlockSpec((NC, BN, H), lambda i: (0, i, 0)),
            pl.BlockSpec((BN, H), lambda i: (i, 0)),
            pl.BlockSpec((BN, 1), lambda i: (i, 0)),
            pl.BlockSpec((H, H), lambda i: (0, 0)),
            pl.BlockSpec((1, H), lambda i: (0, 0)),
        ],
        out_specs=pl.BlockSpec((BN, H), lambda i: (i, 0)),
        out_shape=jax.ShapeDtypeStruct((N, H), jnp.float32),
    )(acc1, y1, dinv, W2p, b1r)


def _fin_body(acc_ref, y2_ref, dinv_ref, b2_ref, out_ref):
    a = acc_ref[0] + acc_ref[1] + y2_ref[...]
    t = dinv_ref[...] * a + b2_ref[...]
    logits = t[:, :C]
    m = jnp.max(logits, axis=1, keepdims=True)
    ex = jnp.exp(logits - m)
    lse = jnp.log(jnp.sum(ex, axis=1, keepdims=True)) + m
    out_ref[...] = logits - lse


@jax.jit
def _fin_kernel(acc2, y2, dinv, b2p):
    return pl.pallas_call(
        _fin_body,
        grid=(pl.cdiv(N, BN),),
        in_specs=[
            pl.BlockSpec((NC, BN, H), lambda i: (0, i, 0)),
            pl.BlockSpec((BN, H), lambda i: (i, 0)),
            pl.BlockSpec((BN, 1), lambda i: (i, 0)),
            pl.BlockSpec((1, H), lambda i: (0, 0)),
        ],
        out_specs=pl.BlockSpec((BN, C), lambda i: (i, 0)),
        out_shape=jax.ShapeDtypeStruct((N, C), jnp.float32),
    )(acc2, y2, dinv, b2p)


def kernel(x, edge_index, W1, b1, W2, b2):
    src_w = edge_index[0].reshape(NW, EW)
    dst_w = edge_index[1].reshape(NW, EW)
    W2p = jnp.pad(W2, ((0, 0), (0, H - C)))
    b1r = b1.reshape(1, H)
    b2p = jnp.pad(b2, (0, H - C)).reshape(1, H)

    deg_parts = _deg_kernel(dst_w)
    dinv, y1 = _prep_kernel(deg_parts, x, W1)
    acc1 = _msg_kernel(y1, src_w, dst_w)
    y2 = _mid_kernel(acc1, y1, dinv, W2p, b1r)
    acc2 = _msg_kernel(y2, src_w, dst_w)
    return _fin_kernel(acc2, y2, dinv, b2p)
